# baseline (speedup 1.0000x reference)
import jax
import jax.numpy as jnp
from jax import lax
from jax.experimental import pallas as pl
from jax.experimental.pallas import tpu as pltpu
from jax.experimental.pallas import tpu_sc as plsc

NU = 25000
NI = 25000
E = 800000
T = 5
IN = 64
HID = 250
OUT = 75

NC = 2
NS = 16
NW = NC * NS
LANES = 16

NKEY = NU * T
NKEY_PAD = 125056
NODES_PAD = 25088
NKEY_HIST = NODES_PAD * T
E_PAD = 819200

EPT = E_PAD // NW
DEG_CH = 1280
DEG_NCH = EPT // DEG_CH
DEG_NVEC = DEG_CH // LANES

NRANGE = 6
RSPAN = 20864
NKEY_OUT = NRANGE * RSPAN
ACC_ROWS = RSPAN + LANES
ZPT = ACC_ROWS // NS
EPS = E_PAD // NS
MCH = 6400
MNCH = EPS // MCH
NBUF = 2
PLIST = MCH + NBUF * 128
DPT = RSPAN // NS

_mesh = plsc.VectorSubcoreMesh(
    core_axis_name="c", subcore_axis_name="s", num_cores=NC, num_subcores=NS)
_sc_params = pltpu.CompilerParams(
    needs_layout_passes=False, use_tc_tiling_on_sc=False)


def _deg_body(src_hbm, dst_hbm, et_hbm, keyu_hbm, keyi_hbm, hu_hbm, hi_hbm,
              hist, nbuf, tbuf, kbuf):
    c = lax.axis_index("c")
    s = lax.axis_index("s")
    wid = c * NS + s
    ebase = wid * EPT
    ones = jnp.ones((LANES,), jnp.float32)

    for node_hbm, key_hbm, h_hbm in ((src_hbm, keyu_hbm, hu_hbm),
                                     (dst_hbm, keyi_hbm, hi_hbm)):
        def zbody(m, carry):
            hist[pl.ds(m * LANES, LANES)] = jnp.zeros((LANES,), jnp.float32)
            return carry
        lax.fori_loop(0, NKEY_HIST // LANES, zbody, 0)

        def cbody(ch, carry):
            base = ebase + ch * DEG_CH
            pltpu.sync_copy(et_hbm.at[pl.ds(base, DEG_CH)], tbuf)
            pltpu.sync_copy(node_hbm.at[pl.ds(base, DEG_CH)], nbuf)

            def sbody(j, carry2):
                o = j * LANES
                tv = tbuf[pl.ds(o, LANES)]
                nv = nbuf[pl.ds(o, LANES)]
                kv = nv * T + tv
                kbuf[pl.ds(o, LANES)] = kv
                plsc.addupdate_scatter(hist, [kv], ones)
                return carry2
            lax.fori_loop(0, DEG_NVEC, sbody, 0)
            pltpu.sync_copy(kbuf, key_hbm.at[pl.ds(base, DEG_CH)])
            return carry
        lax.fori_loop(0, DEG_NCH, cbody, 0)
        pltpu.sync_copy(hist, h_hbm.at[wid])


_deg_kernel = pl.kernel(
    _deg_body,
    out_type=[
        jax.ShapeDtypeStruct((E_PAD,), jnp.int32),
        jax.ShapeDtypeStruct((E_PAD,), jnp.int32),
        jax.ShapeDtypeStruct((NW, NKEY_HIST), jnp.float32),
        jax.ShapeDtypeStruct((NW, NKEY_HIST), jnp.float32),
    ],
    mesh=_mesh,
    scratch_types=[
        pltpu.VMEM((NKEY_HIST,), jnp.float32),
        pltpu.VMEM((DEG_CH,), jnp.int32),
        pltpu.VMEM((DEG_CH,), jnp.int32),
        pltpu.VMEM((DEG_CH,), jnp.int32),
    ],
    compiler_params=_sc_params,
)


def _agg_body(table_hbm, gkey_hbm, skey_hbm, zeros_hbm, out_hbm,
              gk, sk, plist, gidx0, sidx0, gidx1, sidx1,
              rowbuf0, rowbuf1, acc, gsem0, gsem1):
    c = lax.axis_index("c")
    s = lax.axis_index("s")
    gidx = (gidx0, gidx1)
    sidx = (sidx0, sidx1)
    rowbuf = (rowbuf0, rowbuf1)
    gsem = (gsem0, gsem1)
    trash = RSPAN + s
    zvec = jnp.zeros((LANES,), jnp.int32)
    tpad = zvec + (trash << 17)

    for p in range(NRANGE // NC):
        rng = c * (NRANGE // NC) + p
        lo = rng * RSPAN
        pltpu.sync_copy(zeros_hbm, rowbuf0)
        zrow = s * ZPT

        def zchunk(k, carry):
            pltpu.sync_copy(rowbuf0, acc.at[pl.ds(zrow + k * 128, 128)])
            return carry
        lax.fori_loop(0, ZPT // 128, zchunk, 0)
        pltpu.sync_copy(rowbuf0.at[pl.ds(0, ZPT % 128)],
                        acc.at[pl.ds(zrow + (ZPT // 128) * 128, ZPT % 128)])
        plsc.subcore_barrier()

        def subround(q, carry):
            base = s * EPS + q * MCH
            pltpu.sync_copy(gkey_hbm.at[pl.ds(base, MCH)], gk)
            pltpu.sync_copy(skey_hbm.at[pl.ds(base, MCH)], sk)

            def vec_body(j, cnt2):
                o = j * LANES
                kv = sk[pl.ds(o, LANES)]
                gv = gk[pl.ds(o, LANES)]
                inr = (kv >= lo) & (kv < lo + RSPAN)
                packed = gv | ((kv - lo) << 17)
                plsc.store_compressed(plist.at[pl.ds(cnt2, LANES)],
                                      packed, mask=inr)
                pc = plsc.all_reduce_population_count(inr)
                return cnt2 + jnp.max(pc)
            cnt = lax.fori_loop(0, MCH // LANES, vec_body, 0)

            for v in range(NBUF * 128 // LANES):
                plist[pl.ds(cnt + v * LANES, LANES)] = tpad
            ntrip = (cnt + NBUF * 128 - 1) // (NBUF * 128)

            def trip_body(h, carry2):
                g0 = h * (NBUF * 128)
                dg = []
                for b in range(NBUF):
                    for v in range(8):
                        pk = plist[pl.ds(g0 + b * 128 + v * LANES, LANES)]
                        gidx[b][pl.ds(v * LANES, LANES)] = pk & 0x1FFFF
                        sidx[b][pl.ds(v * LANES, LANES)] = (
                            lax.shift_right_logical(pk, 17))
                    dg.append(pltpu.async_copy(
                        table_hbm.at[gidx[b]], rowbuf[b], gsem[b]))
                for b in range(NBUF):
                    dg[b].wait()
                    pltpu.sync_copy(rowbuf[b], acc.at[sidx[b]], add=True)
                return carry2
            lax.fori_loop(0, ntrip, trip_body, 0)
            return carry
        lax.fori_loop(0, MNCH, subround, 0)
        plsc.subcore_barrier()

        dstart = s * DPT
        pltpu.sync_copy(acc.at[pl.ds(dstart, DPT)],
                        out_hbm.at[pl.ds(lo + dstart, DPT)])
        plsc.subcore_barrier()


_agg_kernel = pl.kernel(
    _agg_body,
    out_type=[jax.ShapeDtypeStruct((NKEY_OUT, IN), jnp.float32)],
    mesh=_mesh,
    scratch_types=[
        pltpu.VMEM((MCH,), jnp.int32),
        pltpu.VMEM((MCH,), jnp.int32),
        pltpu.VMEM((PLIST,), jnp.int32),
        pltpu.VMEM((128,), jnp.int32),
        pltpu.VMEM((128,), jnp.int32),
        pltpu.VMEM((128,), jnp.int32),
        pltpu.VMEM((128,), jnp.int32),
        pltpu.VMEM((128, IN), jnp.float32),
        pltpu.VMEM((128, IN), jnp.float32),
        pltpu.VMEM_SHARED((ACC_ROWS, IN), jnp.float32),
        pltpu.SemaphoreType.DMA,
        pltpu.SemaphoreType.DMA,
    ],
    compiler_params=_sc_params,
)

BN = 1000
NBLK = NU // BN


def _leaky(x):
    return jnp.where(x >= 0, x, 0.01 * x)


def _tc_scale_body(parts_ref, feats_ref, scaled_ref, rvec_ref):
    deg = jnp.sum(parts_ref[...], axis=0)
    rv = lax.rsqrt(jnp.maximum(deg, 1.0))
    rvec_ref[...] = rv
    scaled = feats_ref[...][:, None, :] * rv[:, :, None]
    scaled_ref[...] = scaled.reshape(BN * T, IN)


def _tc_scale(parts, feats):
    return pl.pallas_call(
        _tc_scale_body,
        grid=(NBLK,),
        in_specs=[
            pl.BlockSpec((NW, BN, T), lambda b: (0, b, 0)),
            pl.BlockSpec((BN, IN), lambda b: (b, 0)),
        ],
        out_specs=[
            pl.BlockSpec((BN * T, IN), lambda b: (b, 0)),
            pl.BlockSpec((BN, T), lambda b: (b, 0)),
        ],
        out_shape=[
            jax.ShapeDtypeStruct((NKEY, IN), jnp.float32),
            jax.ShapeDtypeStruct((NU, T), jnp.float32),
        ],
    )(parts, feats)


def _tc_out_body(agg_ref, rvec_ref, wr_ref, whw_ref, whb_ref,
                 w1_ref, b1_ref, w2_ref, b2_ref, fh_ref, fr_ref):
    rv = rvec_ref[...]
    agg = agg_ref[...].reshape(BN, T, IN)
    acc = jnp.zeros((BN, HID), jnp.float32)
    for t in range(T):
        x = agg[:, t, :] * rv[:, t][:, None]
        acc = acc + jnp.dot(x, wr_ref[t],
                            preferred_element_type=jnp.float32)
    h = _leaky(acc)
    fh = jnp.dot(h, whw_ref[...],
                 preferred_element_type=jnp.float32) + whb_ref[...]
    fh_ref[...] = fh
    z = _leaky(jnp.dot(fh, w1_ref[...],
                       preferred_element_type=jnp.float32) + b1_ref[...])
    fr_ref[...] = jnp.dot(z, w2_ref[...],
                          preferred_element_type=jnp.float32) + b2_ref[...]


def _full_spec(shape):
    nd = len(shape)
    return pl.BlockSpec(shape, lambda b, _n=nd: (0,) * _n)


def _tc_out(agg, rvec, W_r, W_h_w, W_h_b, W1_w, W1_b, W2_w, W2_b):
    return pl.pallas_call(
        _tc_out_body,
        grid=(NBLK,),
        in_specs=[
            pl.BlockSpec((BN * T, IN), lambda b: (b, 0)),
            pl.BlockSpec((BN, T), lambda b: (b, 0)),
            _full_spec((T, IN, HID)),
            _full_spec((HID, OUT)),
            _full_spec((1, OUT)),
            _full_spec((OUT, IN)),
            _full_spec((1, IN)),
            _full_spec((IN, IN)),
            _full_spec((1, IN)),
        ],
        out_specs=[
            pl.BlockSpec((BN, OUT), lambda b: (b, 0)),
            pl.BlockSpec((BN, IN), lambda b: (b, 0)),
        ],
        out_shape=[
            jax.ShapeDtypeStruct((NU, OUT), jnp.float32),
            jax.ShapeDtypeStruct((NU, IN), jnp.float32),
        ],
    )(agg, rvec, W_r, W_h_w, W_h_b, W1_w, W1_b, W2_w, W2_b)


def kernel(ufeats, ifeats, edge_index, edge_type, W_r, W_h_w, W_h_b,
           W1_w, W1_b, W2_w, W2_b):
    src = edge_index[0].astype(jnp.int32)
    dst = edge_index[1].astype(jnp.int32)
    et = edge_type.astype(jnp.int32)
    npad = E_PAD - E
    src = jnp.concatenate([src, jnp.full((npad,), NU, jnp.int32)])
    dst = jnp.concatenate([dst, jnp.full((npad,), NI, jnp.int32)])
    et = jnp.concatenate([et, jnp.zeros((npad,), jnp.int32)])

    keyU, keyI, huP, hiP = _deg_kernel(src, dst, et)
    Usc, ru = _tc_scale(huP.reshape(NW, NODES_PAD, T), ufeats)
    Isc, ri = _tc_scale(hiP.reshape(NW, NODES_PAD, T), ifeats)

    zeros128 = jnp.zeros((128, IN), jnp.float32)
    A = _agg_kernel(Usc, keyU, keyI, zeros128)
    B = _agg_kernel(Isc, keyI, keyU, zeros128)
    if isinstance(A, (tuple, list)):
        A, = A
        B, = B

    ifeats_h, ifeats_r = _tc_out(A, ri, W_r, W_h_w,
                                 W_h_b.reshape(1, OUT), W1_w,
                                 W1_b.reshape(1, IN), W2_w,
                                 W2_b.reshape(1, IN))
    ufeats_h, ufeats_r = _tc_out(B, ru, W_r, W_h_w,
                                 W_h_b.reshape(1, OUT), W1_w,
                                 W1_b.reshape(1, IN), W2_w,
                                 W2_b.reshape(1, IN))
    return (ufeats_h, ifeats_h, ufeats_r, ifeats_r)


# speedup vs baseline: 1.6615x; 1.6615x over previous
import jax
import jax.numpy as jnp
from jax import lax
from jax.experimental import pallas as pl
from jax.experimental.pallas import tpu as pltpu
from jax.experimental.pallas import tpu_sc as plsc

NU = 25000
NI = 25000
E = 800000
T = 5
IN = 64
HID = 250
OUT = 75

NC = 2
NS = 16
NW = NC * NS
LANES = 16

NKEY = NU * T
NKEY_PAD = 125056
NODES_PAD = 25088
NKEY_HIST = NODES_PAD * T
E_PAD = 819200

EPT = E_PAD // NW
DEG_CH = 1280
DEG_NCH = EPT // DEG_CH
DEG_NVEC = DEG_CH // LANES

NRANGE = 6
RSPAN = 20864
NKEY_OUT = NRANGE * RSPAN
ACC_ROWS = RSPAN + LANES
ZPT = ACC_ROWS // NS
EPS = E_PAD // NS
MCH = 2048
MNCH = EPS // MCH
SUBR = (7, 6, 6, 6)
NBUF = 2
PLIST = SUBR[0] * MCH + NBUF * 128
DPT = RSPAN // NS

_mesh = plsc.VectorSubcoreMesh(
    core_axis_name="c", subcore_axis_name="s", num_cores=NC, num_subcores=NS)
_sc_params = pltpu.CompilerParams(
    needs_layout_passes=False, use_tc_tiling_on_sc=False)


def _deg_body(src_hbm, dst_hbm, et_hbm, keyu_hbm, keyi_hbm, hu_hbm, hi_hbm,
              hist, nbuf, tbuf, kbuf):
    c = lax.axis_index("c")
    s = lax.axis_index("s")
    wid = c * NS + s
    ebase = wid * EPT
    ones = jnp.ones((LANES,), jnp.float32)

    for node_hbm, key_hbm, h_hbm in ((src_hbm, keyu_hbm, hu_hbm),
                                     (dst_hbm, keyi_hbm, hi_hbm)):
        def zbody(m, carry):
            hist[pl.ds(m * LANES, LANES)] = jnp.zeros((LANES,), jnp.float32)
            return carry
        lax.fori_loop(0, NKEY_HIST // LANES, zbody, 0)

        def cbody(ch, carry):
            base = ebase + ch * DEG_CH
            pltpu.sync_copy(et_hbm.at[pl.ds(base, DEG_CH)], tbuf)
            pltpu.sync_copy(node_hbm.at[pl.ds(base, DEG_CH)], nbuf)

            def sbody(j, carry2):
                o = j * LANES
                tv = tbuf[pl.ds(o, LANES)]
                nv = nbuf[pl.ds(o, LANES)]
                kv = nv * T + tv
                kbuf[pl.ds(o, LANES)] = kv
                plsc.addupdate_scatter(hist, [kv], ones)
                return carry2
            lax.fori_loop(0, DEG_NVEC, sbody, 0)
            pltpu.sync_copy(kbuf, key_hbm.at[pl.ds(base, DEG_CH)])
            return carry
        lax.fori_loop(0, DEG_NCH, cbody, 0)
        pltpu.sync_copy(hist, h_hbm.at[wid])


_deg_kernel = pl.kernel(
    _deg_body,
    out_type=[
        jax.ShapeDtypeStruct((E_PAD,), jnp.int32),
        jax.ShapeDtypeStruct((E_PAD,), jnp.int32),
        jax.ShapeDtypeStruct((NW, NKEY_HIST), jnp.float32),
        jax.ShapeDtypeStruct((NW, NKEY_HIST), jnp.float32),
    ],
    mesh=_mesh,
    scratch_types=[
        pltpu.VMEM((NKEY_HIST,), jnp.float32),
        pltpu.VMEM((DEG_CH,), jnp.int32),
        pltpu.VMEM((DEG_CH,), jnp.int32),
        pltpu.VMEM((DEG_CH,), jnp.int32),
    ],
    compiler_params=_sc_params,
)


def _agg_body(table_hbm, gkey_hbm, skey_hbm, zeros_hbm, out_hbm,
              gk, sk, plist, gidx0, sidx0, gidx1, sidx1,
              rowbuf0, rowbuf1, zbuf, acc, gsem0, gsem1):
    c = lax.axis_index("c")
    s = lax.axis_index("s")
    gidx = (gidx0, gidx1)
    sidx = (sidx0, sidx1)
    rowbuf = (rowbuf0, rowbuf1)
    gsem = (gsem0, gsem1)
    trash = RSPAN + s
    zvec = jnp.zeros((LANES,), jnp.int32)
    tpad = zvec + (trash << 17)

    pltpu.sync_copy(zeros_hbm, zbuf)
    for p in range(NRANGE // NC):
        rng = c * (NRANGE // NC) + p
        lo = rng * RSPAN
        zrow = s * ZPT

        def zchunk(k, carry):
            pltpu.sync_copy(zbuf, acc.at[pl.ds(zrow + k * 128, 128)])
            return carry
        lax.fori_loop(0, ZPT // 128, zchunk, 0)
        pltpu.sync_copy(zbuf.at[pl.ds(0, ZPT % 128)],
                        acc.at[pl.ds(zrow + (ZPT // 128) * 128, ZPT % 128)])
        plsc.subcore_barrier()

        ch0 = 0
        for qn in SUBR:
            def chunk_body(ch, cnt, _ch0=ch0):
                base = s * EPS + (_ch0 + ch) * MCH
                pltpu.sync_copy(gkey_hbm.at[pl.ds(base, MCH)], gk)
                pltpu.sync_copy(skey_hbm.at[pl.ds(base, MCH)], sk)

                def vec_body(j, cnt2):
                    o = j * LANES
                    kv = sk[pl.ds(o, LANES)]
                    gv = gk[pl.ds(o, LANES)]
                    inr = (kv >= lo) & (kv < lo + RSPAN)
                    packed = gv | ((kv - lo) << 17)
                    plsc.store_compressed(plist.at[pl.ds(cnt2, LANES)],
                                          packed, mask=inr)
                    pc = plsc.all_reduce_population_count(inr)
                    return cnt2 + jnp.max(pc)
                return lax.fori_loop(0, MCH // LANES, vec_body, cnt)
            cnt = lax.fori_loop(0, qn, chunk_body, 0)
            ch0 += qn

            for v in range(NBUF * 128 // LANES):
                plist[pl.ds(cnt + v * LANES, LANES)] = tpad
            ntrip = (cnt + NBUF * 128 - 1) // (NBUF * 128)

            def trip_body(h, carry2):
                g0 = h * (NBUF * 128)
                dg = []
                for b in range(NBUF):
                    for v in range(8):
                        pk = plist[pl.ds(g0 + b * 128 + v * LANES, LANES)]
                        gidx[b][pl.ds(v * LANES, LANES)] = pk & 0x1FFFF
                        sidx[b][pl.ds(v * LANES, LANES)] = (
                            lax.shift_right_logical(pk, 17))
                    dg.append(pltpu.async_copy(
                        table_hbm.at[gidx[b]], rowbuf[b], gsem[b]))
                for b in range(NBUF):
                    dg[b].wait()
                    pltpu.sync_copy(rowbuf[b], acc.at[sidx[b]], add=True)
                return carry2
            lax.fori_loop(0, ntrip, trip_body, 0)
        plsc.subcore_barrier()

        dstart = s * DPT
        pltpu.sync_copy(acc.at[pl.ds(dstart, DPT)],
                        out_hbm.at[pl.ds(lo + dstart, DPT)])
        plsc.subcore_barrier()


_agg_kernel = pl.kernel(
    _agg_body,
    out_type=[jax.ShapeDtypeStruct((NKEY_OUT, IN), jnp.float32)],
    mesh=_mesh,
    scratch_types=[
        pltpu.VMEM((MCH,), jnp.int32),
        pltpu.VMEM((MCH,), jnp.int32),
        pltpu.VMEM((PLIST,), jnp.int32),
        pltpu.VMEM((128,), jnp.int32),
        pltpu.VMEM((128,), jnp.int32),
        pltpu.VMEM((128,), jnp.int32),
        pltpu.VMEM((128,), jnp.int32),
        pltpu.VMEM((128, IN), jnp.float32),
        pltpu.VMEM((128, IN), jnp.float32),
        pltpu.VMEM((128, IN), jnp.float32),
        pltpu.VMEM_SHARED((ACC_ROWS, IN), jnp.float32),
        pltpu.SemaphoreType.DMA,
        pltpu.SemaphoreType.DMA,
    ],
    compiler_params=_sc_params,
)

BN = 1000
NBLK = NU // BN


def _leaky(x):
    return jnp.where(x >= 0, x, 0.01 * x)


def _tc_scale_body(parts_ref, feats_ref, scaled_ref, rvec_ref):
    deg = jnp.sum(parts_ref[...], axis=0)
    rv = lax.rsqrt(jnp.maximum(deg, 1.0))
    rvec_ref[...] = rv
    scaled = feats_ref[...][:, None, :] * rv[:, :, None]
    scaled_ref[...] = scaled.reshape(BN * T, IN)


def _tc_scale(parts, feats):
    return pl.pallas_call(
        _tc_scale_body,
        grid=(NBLK,),
        in_specs=[
            pl.BlockSpec((NW, BN, T), lambda b: (0, b, 0)),
            pl.BlockSpec((BN, IN), lambda b: (b, 0)),
        ],
        out_specs=[
            pl.BlockSpec((BN * T, IN), lambda b: (b, 0)),
            pl.BlockSpec((BN, T), lambda b: (b, 0)),
        ],
        out_shape=[
            jax.ShapeDtypeStruct((NKEY, IN), jnp.float32),
            jax.ShapeDtypeStruct((NU, T), jnp.float32),
        ],
    )(parts, feats)


def _tc_out_body(agg_ref, rvec_ref, wr_ref, whw_ref, whb_ref,
                 w1_ref, b1_ref, w2_ref, b2_ref, fh_ref, fr_ref):
    rv = rvec_ref[...]
    agg = agg_ref[...].reshape(BN, T, IN)
    acc = jnp.zeros((BN, HID), jnp.float32)
    for t in range(T):
        x = agg[:, t, :] * rv[:, t][:, None]
        acc = acc + jnp.dot(x, wr_ref[t],
                            preferred_element_type=jnp.float32)
    h = _leaky(acc)
    fh = jnp.dot(h, whw_ref[...],
                 preferred_element_type=jnp.float32) + whb_ref[...]
    fh_ref[...] = fh
    z = _leaky(jnp.dot(fh, w1_ref[...],
                       preferred_element_type=jnp.float32) + b1_ref[...])
    fr_ref[...] = jnp.dot(z, w2_ref[...],
                          preferred_element_type=jnp.float32) + b2_ref[...]


def _full_spec(shape):
    nd = len(shape)
    return pl.BlockSpec(shape, lambda b, _n=nd: (0,) * _n)


def _tc_out(agg, rvec, W_r, W_h_w, W_h_b, W1_w, W1_b, W2_w, W2_b):
    return pl.pallas_call(
        _tc_out_body,
        grid=(NBLK,),
        in_specs=[
            pl.BlockSpec((BN * T, IN), lambda b: (b, 0)),
            pl.BlockSpec((BN, T), lambda b: (b, 0)),
            _full_spec((T, IN, HID)),
            _full_spec((HID, OUT)),
            _full_spec((1, OUT)),
            _full_spec((OUT, IN)),
            _full_spec((1, IN)),
            _full_spec((IN, IN)),
            _full_spec((1, IN)),
        ],
        out_specs=[
            pl.BlockSpec((BN, OUT), lambda b: (b, 0)),
            pl.BlockSpec((BN, IN), lambda b: (b, 0)),
        ],
        out_shape=[
            jax.ShapeDtypeStruct((NU, OUT), jnp.float32),
            jax.ShapeDtypeStruct((NU, IN), jnp.float32),
        ],
    )(agg, rvec, W_r, W_h_w, W_h_b, W1_w, W1_b, W2_w, W2_b)


def kernel(ufeats, ifeats, edge_index, edge_type, W_r, W_h_w, W_h_b,
           W1_w, W1_b, W2_w, W2_b):
    src = edge_index[0].astype(jnp.int32)
    dst = edge_index[1].astype(jnp.int32)
    et = edge_type.astype(jnp.int32)
    npad = E_PAD - E
    src = jnp.concatenate([src, jnp.full((npad,), NU, jnp.int32)])
    dst = jnp.concatenate([dst, jnp.full((npad,), NI, jnp.int32)])
    et = jnp.concatenate([et, jnp.zeros((npad,), jnp.int32)])

    keyU, keyI, huP, hiP = _deg_kernel(src, dst, et)
    Usc, ru = _tc_scale(huP.reshape(NW, NODES_PAD, T), ufeats)
    Isc, ri = _tc_scale(hiP.reshape(NW, NODES_PAD, T), ifeats)

    zeros128 = jnp.zeros((128, IN), jnp.float32)
    A = _agg_kernel(Usc, keyU, keyI, zeros128)
    B = _agg_kernel(Isc, keyI, keyU, zeros128)
    if isinstance(A, (tuple, list)):
        A, = A
        B, = B

    ifeats_h, ifeats_r = _tc_out(A, ri, W_r, W_h_w,
                                 W_h_b.reshape(1, OUT), W1_w,
                                 W1_b.reshape(1, IN), W2_w,
                                 W2_b.reshape(1, IN))
    ufeats_h, ufeats_r = _tc_out(B, ru, W_r, W_h_w,
                                 W_h_b.reshape(1, OUT), W1_w,
                                 W1_b.reshape(1, IN), W2_w,
                                 W2_b.reshape(1, IN))
    return (ufeats_h, ifeats_h, ufeats_r, ifeats_r)
